# baseline (speedup 1.0000x reference)
import functools

import jax
import jax.numpy as jnp
from jax import lax
from jax.experimental import pallas as pl
from jax.experimental.pallas import tpu as pltpu
from jax.experimental.pallas import tpu_sc as plsc

B = 4096
K = 4
S = 64
D = 32
SD = S * D
TEMPERATURE = 0.2
LANES = 16

NC = 2
NS = 16
NW = NC * NS
RPW = B // NW
SB = 8
NSB = RPW // SB
PPW = SD // NW
PB = 4
NPB = PPW // PB
NG = B // LANES


def _weights_body(sim_hbm, idxt_hbm, wt_hbm,
                  idxt_v, svals_v, wt_v, simbuf, sems0, sems1):
    wid = lax.axis_index("s") * NC + lax.axis_index("c")
    base = wid * RPW
    lane = jnp.arange(LANES, dtype=jnp.int32)

    pltpu.sync_copy(idxt_hbm.at[:, pl.ds(base, RPW)], idxt_v)

    sems = (sems0, sems1)

    def start_sim(sb, b):
        pltpu.async_copy(sim_hbm.at[pl.ds(base + sb * SB, SB)],
                         simbuf.at[b], sems[b])

    def wait_sim(b):
        pltpu.make_async_copy(sim_hbm.at[pl.ds(base, SB)],
                              simbuf.at[b], sems[b]).wait()

    start_sim(0, 0)
    start_sim(1, 1)
    rows4 = lane >> 2
    kof = lane & 3

    @pl.loop(0, NSB // 2)
    def sim_body(i):
        for b in range(2):
            sb = 2 * i + b
            wait_sim(b)
            for h in range(SB * K // LANES):
                rloc = h * (LANES // K) + rows4
                cols = plsc.load_gather(idxt_v, [kof, sb * SB + rloc])
                svals_v[pl.ds((sb * 2 + h) * LANES, LANES)] = plsc.load_gather(
                    simbuf.at[b], [rloc, cols])

            @pl.when(sb + 2 < NSB)
            def _():
                start_sim(sb + 2, b)

    inv_t = jnp.float32(1.0) / jnp.float32(TEMPERATURE)
    for t in range(RPW // LANES):
        ridx0 = (t * LANES + lane) * K
        s = [plsc.load_gather(svals_v, [ridx0 + k]) * inv_t for k in range(K)]
        m = jnp.maximum(jnp.maximum(s[0], s[1]), jnp.maximum(s[2], s[3]))
        e = [jnp.exp(sk - m) for sk in s]
        den = (e[0] + e[1]) + (e[2] + e[3])
        for k in range(K):
            wt_v[k, pl.ds(t * LANES, LANES)] = e[k] / den

    pltpu.sync_copy(wt_v, wt_hbm.at[:, pl.ds(base, RPW)])


def _rebuild_body(embt_hbm, idxt_hbm, wt_hbm, outt_hbm,
                  idxb, wb, ebuf, obuf, seme0, seme1, semo0, semo1):
    wid = lax.axis_index("s") * NC + lax.axis_index("c")
    pbase = wid * PPW

    pltpu.sync_copy(idxt_hbm, idxb)
    pltpu.sync_copy(wt_hbm, wb)

    seme = (seme0, seme1)
    semo = (semo0, semo1)

    def start_feat(t, b):
        for p in range(PB):
            pltpu.async_copy(embt_hbm.at[pbase + t * PB + p],
                             ebuf.at[pl.ds((b * PB + p) * B, B)], seme[b])

    def wait_feat(b):
        for p in range(PB):
            pltpu.make_async_copy(embt_hbm.at[pbase + p],
                                  ebuf.at[pl.ds((b * PB + p) * B, B)],
                                  seme[b]).wait()

    def wait_out(b):
        pltpu.make_async_copy(obuf.at[b], outt_hbm.at[pl.ds(pbase, PB)],
                              semo[b]).wait()

    start_feat(0, 0)
    start_feat(1, 1)

    @pl.loop(0, NPB // 2)
    def feat_pair(i):
        for b in range(2):
            t = 2 * i + b
            wait_feat(b)

            @pl.when(t >= 2)
            def _():
                wait_out(b)

            @plsc.parallel_loop(0, NG, unroll=4)
            def g_body(g):
                sl = pl.ds(g * LANES, LANES)
                ik = [idxb[k, sl] for k in range(K)]
                wk = [wb[k, sl] for k in range(K)]
                for p in range(PB):
                    off = jnp.int32((b * PB + p) * B)
                    v = [plsc.load_gather(ebuf, [ik[k] | off])
                         for k in range(K)]
                    acc = wk[0] * v[0]
                    for k in range(1, K):
                        acc = acc + wk[k] * v[k]
                    obuf[b, p, sl] = acc

            pltpu.async_copy(obuf.at[b],
                             outt_hbm.at[pl.ds(pbase + t * PB, PB)],
                             semo[b])

            @pl.when(t + 2 < NPB)
            def _():
                start_feat(t + 2, b)

    wait_out(0)
    wait_out(1)


def kernel(similarity_matrix, batch_emb_om, index):
    embt = batch_emb_om.transpose(1, 2, 0).reshape(SD, B)
    idxt = index.T

    mesh = plsc.VectorSubcoreMesh(core_axis_name="c", subcore_axis_name="s",
                                  num_cores=NC, num_subcores=NS)
    ka = pl.kernel(
        _weights_body,
        out_type=jax.ShapeDtypeStruct((K, B), jnp.float32),
        mesh=mesh,
        scratch_types=[
            pltpu.VMEM((K, RPW), jnp.int32),
            pltpu.VMEM((RPW * K,), jnp.float32),
            pltpu.VMEM((K, RPW), jnp.float32),
            pltpu.VMEM((2, SB, B), jnp.float32),
            pltpu.SemaphoreType.DMA,
            pltpu.SemaphoreType.DMA,
        ],
        compiler_params=pltpu.CompilerParams(needs_layout_passes=False),
    )
    wt = ka(similarity_matrix, idxt)

    kb = pl.kernel(
        _rebuild_body,
        out_type=jax.ShapeDtypeStruct((SD, B), jnp.float32),
        mesh=mesh,
        scratch_types=[
            pltpu.VMEM((K, B), jnp.int32),
            pltpu.VMEM((K, B), jnp.float32),
            pltpu.VMEM((2 * PB * B,), jnp.float32),
            pltpu.VMEM((2, PB, B), jnp.float32),
            pltpu.SemaphoreType.DMA,
            pltpu.SemaphoreType.DMA,
            pltpu.SemaphoreType.DMA,
            pltpu.SemaphoreType.DMA,
        ],
        compiler_params=pltpu.CompilerParams(needs_layout_passes=False),
    )
    outt = kb(embt, idxt, wt)

    out = outt.reshape(S, D, B).transpose(2, 0, 1)
    return (wt.T, out)


# speedup vs baseline: 1.1424x; 1.1424x over previous
import functools

import jax
import jax.numpy as jnp
from jax import lax
from jax.experimental import pallas as pl
from jax.experimental.pallas import tpu as pltpu
from jax.experimental.pallas import tpu_sc as plsc

B = 4096
K = 4
S = 64
D = 32
SD = S * D
TEMPERATURE = 0.2
LANES = 16

NC = 2
NS = 16
NW = NC * NS
RPW = B // NW
SB = 8
NSB = RPW // SB
PPW = SD // NW
PB = 4
NPB = PPW // PB
NG = B // LANES


def _weights_body(sim_hbm, idxt_hbm, wt_hbm,
                  idxt_v, svals_v, wt_v, simbuf, sems0, sems1):
    wid = lax.axis_index("s") * NC + lax.axis_index("c")
    base = wid * RPW
    lane = jnp.arange(LANES, dtype=jnp.int32)

    pltpu.sync_copy(idxt_hbm.at[:, pl.ds(base, RPW)], idxt_v)

    sems = (sems0, sems1)

    def start_sim(sb, b):
        pltpu.async_copy(sim_hbm.at[pl.ds(base + sb * SB, SB)],
                         simbuf.at[b], sems[b])

    def wait_sim(b):
        pltpu.make_async_copy(sim_hbm.at[pl.ds(base, SB)],
                              simbuf.at[b], sems[b]).wait()

    start_sim(0, 0)
    start_sim(1, 1)
    rows4 = lane >> 2
    kof = lane & 3

    @pl.loop(0, NSB // 2)
    def sim_body(i):
        for b in range(2):
            sb = 2 * i + b
            wait_sim(b)
            for h in range(SB * K // LANES):
                rloc = h * (LANES // K) + rows4
                cols = plsc.load_gather(idxt_v, [kof, sb * SB + rloc])
                svals_v[pl.ds((sb * 2 + h) * LANES, LANES)] = plsc.load_gather(
                    simbuf.at[b], [rloc, cols])

            @pl.when(sb + 2 < NSB)
            def _():
                start_sim(sb + 2, b)

    inv_t = jnp.float32(1.0) / jnp.float32(TEMPERATURE)
    for t in range(RPW // LANES):
        ridx0 = (t * LANES + lane) * K
        s = [plsc.load_gather(svals_v, [ridx0 + k]) * inv_t for k in range(K)]
        m = jnp.maximum(jnp.maximum(s[0], s[1]), jnp.maximum(s[2], s[3]))
        e = [jnp.exp(sk - m) for sk in s]
        den = (e[0] + e[1]) + (e[2] + e[3])
        for k in range(K):
            wt_v[k, pl.ds(t * LANES, LANES)] = e[k] / den

    pltpu.sync_copy(wt_v, wt_hbm.at[:, pl.ds(base, RPW)])


def _rebuild_body(embt_hbm, idxt_hbm, wt_hbm, outt_hbm,
                  idxb, wb, ebuf, obuf, seme0, seme1, semo0, semo1):
    wid = lax.axis_index("s") * NC + lax.axis_index("c")
    pbase = wid * PPW

    pltpu.sync_copy(idxt_hbm, idxb)
    pltpu.sync_copy(wt_hbm, wb)

    seme = (seme0, seme1)
    semo = (semo0, semo1)

    def start_feat(t, b):
        pltpu.async_copy(embt_hbm.at[pl.ds(pbase + t * PB, PB)],
                         ebuf.at[b], seme[b])

    def wait_feat(b):
        pltpu.make_async_copy(embt_hbm.at[pl.ds(pbase, PB)],
                              ebuf.at[b], seme[b]).wait()

    def wait_out(b):
        pltpu.make_async_copy(obuf.at[b], outt_hbm.at[pl.ds(pbase, PB)],
                              semo[b]).wait()

    start_feat(0, 0)
    start_feat(1, 1)

    @pl.loop(0, NPB // 2)
    def feat_pair(i):
        for b in range(2):
            t = 2 * i + b
            wait_feat(b)

            @pl.when(t >= 2)
            def _():
                wait_out(b)

            bvec = jnp.full((LANES,), b, dtype=jnp.int32)
            pvecs = [jnp.full((LANES,), p, dtype=jnp.int32) for p in range(PB)]

            @plsc.parallel_loop(0, NG, unroll=4)
            def g_body(g):
                sl = pl.ds(g * LANES, LANES)
                ik = [idxb[k, sl] for k in range(K)]
                wk = [wb[k, sl] for k in range(K)]
                for p in range(PB):
                    v = [plsc.load_gather(ebuf, [bvec, pvecs[p], ik[k]])
                         for k in range(K)]
                    acc = wk[0] * v[0]
                    for k in range(1, K):
                        acc = acc + wk[k] * v[k]
                    obuf[b, p, sl] = acc

            pltpu.async_copy(obuf.at[b],
                             outt_hbm.at[pl.ds(pbase + t * PB, PB)],
                             semo[b])

            @pl.when(t + 2 < NPB)
            def _():
                start_feat(t + 2, b)

    wait_out(0)
    wait_out(1)


def kernel(similarity_matrix, batch_emb_om, index):
    embt = batch_emb_om.transpose(1, 2, 0).reshape(SD, B)
    idxt = index.T

    mesh = plsc.VectorSubcoreMesh(core_axis_name="c", subcore_axis_name="s",
                                  num_cores=NC, num_subcores=NS)
    ka = pl.kernel(
        _weights_body,
        out_type=jax.ShapeDtypeStruct((K, B), jnp.float32),
        mesh=mesh,
        scratch_types=[
            pltpu.VMEM((K, RPW), jnp.int32),
            pltpu.VMEM((RPW * K,), jnp.float32),
            pltpu.VMEM((K, RPW), jnp.float32),
            pltpu.VMEM((2, SB, B), jnp.float32),
            pltpu.SemaphoreType.DMA,
            pltpu.SemaphoreType.DMA,
        ],
        compiler_params=pltpu.CompilerParams(needs_layout_passes=False),
    )
    wt = ka(similarity_matrix, idxt)

    kb = pl.kernel(
        _rebuild_body,
        out_type=jax.ShapeDtypeStruct((SD, B), jnp.float32),
        mesh=mesh,
        scratch_types=[
            pltpu.VMEM((K, B), jnp.int32),
            pltpu.VMEM((K, B), jnp.float32),
            pltpu.VMEM((2, PB, B), jnp.float32),
            pltpu.VMEM((2, PB, B), jnp.float32),
            pltpu.SemaphoreType.DMA,
            pltpu.SemaphoreType.DMA,
            pltpu.SemaphoreType.DMA,
            pltpu.SemaphoreType.DMA,
        ],
        compiler_params=pltpu.CompilerParams(needs_layout_passes=False),
    )
    outt = kb(embt, idxt, wt)

    out = outt.reshape(S, D, B).transpose(2, 0, 1)
    return (wt.T, out)


# speedup vs baseline: 1.2958x; 1.1343x over previous
import functools

import jax
import jax.numpy as jnp
from jax import lax
from jax.experimental import pallas as pl
from jax.experimental.pallas import tpu as pltpu
from jax.experimental.pallas import tpu_sc as plsc

B = 4096
K = 4
S = 64
D = 32
SD = S * D
TEMPERATURE = 0.2
LANES = 16

NC = 2
NS = 16
NW = NC * NS
RPW = B // NW
SB = 8
NSB = RPW // SB
PPW = SD // NW
PB = 4
NPB = PPW // PB
NG = B // LANES


def _weights_body(sim_hbm, idxt_hbm, wt_hbm,
                  idxt_v, svals_v, wt_v, simbuf, sems0, sems1):
    wid = lax.axis_index("s") * NC + lax.axis_index("c")
    base = wid * RPW
    lane = jnp.arange(LANES, dtype=jnp.int32)

    pltpu.sync_copy(idxt_hbm.at[:, pl.ds(base, RPW)], idxt_v)

    sems = (sems0, sems1)

    def start_sim(sb, b):
        pltpu.async_copy(sim_hbm.at[pl.ds(base + sb * SB, SB)],
                         simbuf.at[b], sems[b])

    def wait_sim(b):
        pltpu.make_async_copy(sim_hbm.at[pl.ds(base, SB)],
                              simbuf.at[b], sems[b]).wait()

    start_sim(0, 0)
    start_sim(1, 1)
    rows4 = lane >> 2
    kof = lane & 3

    @pl.loop(0, NSB // 2)
    def sim_body(i):
        for b in range(2):
            sb = 2 * i + b
            wait_sim(b)
            for h in range(SB * K // LANES):
                rloc = h * (LANES // K) + rows4
                cols = plsc.load_gather(idxt_v, [kof, sb * SB + rloc])
                svals_v[pl.ds((sb * 2 + h) * LANES, LANES)] = plsc.load_gather(
                    simbuf.at[b], [rloc, cols])

            @pl.when(sb + 2 < NSB)
            def _():
                start_sim(sb + 2, b)

    inv_t = jnp.float32(1.0) / jnp.float32(TEMPERATURE)
    for t in range(RPW // LANES):
        ridx0 = (t * LANES + lane) * K
        s = [plsc.load_gather(svals_v, [ridx0 + k]) * inv_t for k in range(K)]
        m = jnp.maximum(jnp.maximum(s[0], s[1]), jnp.maximum(s[2], s[3]))
        e = [jnp.exp(sk - m) for sk in s]
        den = (e[0] + e[1]) + (e[2] + e[3])
        for k in range(K):
            wt_v[k, pl.ds(t * LANES, LANES)] = e[k] / den

    pltpu.sync_copy(wt_v, wt_hbm.at[:, pl.ds(base, RPW)])


def _rebuild_body(embt_hbm, idxt_hbm, wt_hbm, outt_hbm,
                  idxb, wb, ebuf, obuf, seme0, seme1, semo0, semo1):
    wid = lax.axis_index("s") * NC + lax.axis_index("c")
    pbase = wid * PPW

    pltpu.sync_copy(idxt_hbm, idxb)
    pltpu.sync_copy(wt_hbm, wb)

    seme = (seme0, seme1)
    semo = (semo0, semo1)

    def start_feat(t, b):
        pltpu.async_copy(embt_hbm.at[pl.ds(pbase + t * PB, PB)],
                         ebuf.at[b], seme[b])

    def wait_feat(b):
        pltpu.make_async_copy(embt_hbm.at[pl.ds(pbase, PB)],
                              ebuf.at[b], seme[b]).wait()

    def wait_out(b):
        pltpu.make_async_copy(obuf.at[b], outt_hbm.at[pl.ds(pbase, PB)],
                              semo[b]).wait()

    start_feat(0, 0)
    start_feat(1, 1)

    @pl.loop(0, NPB // 2)
    def feat_pair(i):
        for b in range(2):
            t = 2 * i + b
            wait_feat(b)

            @pl.when(t >= 2)
            def _():
                wait_out(b)

            bvec = jnp.full((LANES,), b, dtype=jnp.int32)
            pvecs = [jnp.full((LANES,), p, dtype=jnp.int32) for p in range(PB)]

            @plsc.parallel_loop(0, NG, unroll=2)
            def g_body(g):
                sl = pl.ds(g * LANES, LANES)
                ik = [idxb[k, sl] for k in range(K)]
                wk = [wb[k, sl] for k in range(K)]
                for p in range(PB):
                    v = [plsc.load_gather(ebuf, [bvec, pvecs[p], ik[k]])
                         for k in range(K)]
                    acc = wk[0] * v[0]
                    for k in range(1, K):
                        acc = acc + wk[k] * v[k]
                    obuf[b, p, sl] = acc

            pltpu.async_copy(obuf.at[b],
                             outt_hbm.at[pl.ds(pbase + t * PB, PB)],
                             semo[b])

            @pl.when(t + 2 < NPB)
            def _():
                start_feat(t + 2, b)

    wait_out(0)
    wait_out(1)


def kernel(similarity_matrix, batch_emb_om, index):
    embt = batch_emb_om.transpose(1, 2, 0).reshape(SD, B)
    idxt = index.T

    mesh = plsc.VectorSubcoreMesh(core_axis_name="c", subcore_axis_name="s",
                                  num_cores=NC, num_subcores=NS)
    ka = pl.kernel(
        _weights_body,
        out_type=jax.ShapeDtypeStruct((K, B), jnp.float32),
        mesh=mesh,
        scratch_types=[
            pltpu.VMEM((K, RPW), jnp.int32),
            pltpu.VMEM((RPW * K,), jnp.float32),
            pltpu.VMEM((K, RPW), jnp.float32),
            pltpu.VMEM((2, SB, B), jnp.float32),
            pltpu.SemaphoreType.DMA,
            pltpu.SemaphoreType.DMA,
        ],
        compiler_params=pltpu.CompilerParams(needs_layout_passes=False),
    )
    wt = ka(similarity_matrix, idxt)

    kb = pl.kernel(
        _rebuild_body,
        out_type=jax.ShapeDtypeStruct((SD, B), jnp.float32),
        mesh=mesh,
        scratch_types=[
            pltpu.VMEM((K, B), jnp.int32),
            pltpu.VMEM((K, B), jnp.float32),
            pltpu.VMEM((2, PB, B), jnp.float32),
            pltpu.VMEM((2, PB, B), jnp.float32),
            pltpu.SemaphoreType.DMA,
            pltpu.SemaphoreType.DMA,
            pltpu.SemaphoreType.DMA,
            pltpu.SemaphoreType.DMA,
        ],
        compiler_params=pltpu.CompilerParams(needs_layout_passes=False),
    )
    outt = kb(embt, idxt, wt)

    out = outt.reshape(S, D, B).transpose(2, 0, 1)
    return (wt.T, out)
